# baseline (speedup 1.0000x reference)
import jax
import jax.numpy as jnp
from jax import lax
from jax.experimental import pallas as pl
from jax.experimental.pallas import tpu as pltpu
from jax.experimental.pallas import tpu_sc as plsc

HIDDEN = 1024
N = 4 * 4096
NC, NS, L = 2, 16, 16
NW = NC * NS
RPW = N // NW
C = 16
NCHUNK = RPW // C
D = 3


def _emb_body(w_hbm, p_hbm, wi_hbm, pi_hbm, o_hbm,
              widx, pidx,
              wb0, wb1, wb2, pb0, pb1, pb2, obuf,
              sw0, sw1, sw2, sp0, sp1, sp2, sem_s):
    wbuf = (wb0, wb1, wb2)
    pbuf = (pb0, pb1, pb2)
    sem_w = (sw0, sw1, sw2)
    sem_p = (sp0, sp1, sp2)

    wid = lax.axis_index("s") * NC + lax.axis_index("c")
    base = wid * RPW
    pltpu.sync_copy(wi_hbm.at[pl.ds(base, RPW)], widx)
    pltpu.sync_copy(pi_hbm.at[pl.ds(base, RPW)], pidx)

    def gather_copies(g, b):
        cw = pltpu.make_async_copy(
            w_hbm.at[widx.at[pl.ds(g * C, C)]], wbuf[b], sem_w[b])
        cp = pltpu.make_async_copy(
            p_hbm.at[pidx.at[pl.ds(g * C, C)]], pbuf[b], sem_p[b])
        return cw, cp

    def store_copy(g):
        return pltpu.make_async_copy(
            obuf, o_hbm.at[pl.ds(base + g * C, C)], sem_s)

    for b in range(D):
        cw, cp = gather_copies(b, b)
        cw.start()
        cp.start()

    PAD = NCHUNK + (-NCHUNK) % D

    @pl.loop(0, PAD, step=D)
    def _tri(g):
        for b in range(D):
            gg = g + b

            @pl.when(gg < NCHUNK)
            def _phase():
                cw, cp = gather_copies(gg, b)
                cw.wait()
                cp.wait()

                @pl.when(gg >= 1)
                def _():
                    store_copy(gg - 1).wait()

                @pl.loop(0, C)
                def _row(r):
                    for u in range(HIDDEN // L):
                        s = pl.ds(u * L, L)
                        obuf[r, s] = wbuf[b][r, s] + pbuf[b][r, s]

                store_copy(gg).start()

                @pl.when(gg + D < NCHUNK)
                def _():
                    nw, np_ = gather_copies(gg + D, b)
                    nw.start()
                    np_.start()

    store_copy(NCHUNK - 1).wait()


def kernel(input_ids, position_ids, word_embeddings, position_embeddings):
    wids = input_ids.reshape(-1).astype(jnp.int32)
    pids = position_ids.reshape(-1).astype(jnp.int32)
    mesh = plsc.VectorSubcoreMesh(core_axis_name="c", subcore_axis_name="s")
    k = pl.kernel(
        _emb_body,
        out_type=jax.ShapeDtypeStruct((N, HIDDEN), jnp.float32),
        mesh=mesh,
        scratch_types=(
            [pltpu.VMEM((RPW,), jnp.int32)] * 2
            + [pltpu.VMEM((C, HIDDEN), jnp.float32)] * (2 * D + 1)
            + [pltpu.SemaphoreType.DMA] * (2 * D + 1)
        ),
    )
    out = k(word_embeddings, position_embeddings, wids, pids)
    return out.reshape(input_ids.shape + (HIDDEN,))


# speedup vs baseline: 1.0448x; 1.0448x over previous
import jax
import jax.numpy as jnp
from jax import lax
from jax.experimental import pallas as pl
from jax.experimental.pallas import tpu as pltpu
from jax.experimental.pallas import tpu_sc as plsc

HIDDEN = 1024
N = 4 * 4096
NC, NS, L = 2, 16, 16
NW = NC * NS
RPW = N // NW
C = 16
NCHUNK = RPW // C
D = 3


def _emb_body(w_hbm, p_hbm, wi_hbm, pi_hbm, o_hbm,
              widx, pidx,
              wb0, wb1, wb2, pb0, pb1, pb2, ob0, ob1,
              sw0, sw1, sw2, sp0, sp1, sp2, ss0, ss1):
    wbuf = (wb0, wb1, wb2)
    pbuf = (pb0, pb1, pb2)
    obuf = (ob0, ob1)
    sem_w = (sw0, sw1, sw2)
    sem_p = (sp0, sp1, sp2)
    sem_s = (ss0, ss1)

    wid = lax.axis_index("s") * NC + lax.axis_index("c")
    base = wid * RPW
    pltpu.sync_copy(wi_hbm.at[pl.ds(base, RPW)], widx)
    pltpu.sync_copy(pi_hbm.at[pl.ds(base, RPW)], pidx)

    def gather_copies(g, b):
        cw = pltpu.make_async_copy(
            w_hbm.at[widx.at[pl.ds(g * C, C)]], wbuf[b], sem_w[b])
        cp = pltpu.make_async_copy(
            p_hbm.at[pidx.at[pl.ds(g * C, C)]], pbuf[b], sem_p[b])
        return cw, cp

    HH = C // 2

    def store_copy(g, h):
        return pltpu.make_async_copy(
            obuf[h], o_hbm.at[pl.ds(base + g * C + h * HH, HH)], sem_s[h])

    for b in range(D):
        cw, cp = gather_copies(b, b)
        cw.start()
        cp.start()

    PAD = NCHUNK + (-NCHUNK) % D

    @pl.loop(0, PAD, step=D)
    def _tri(g):
        for b in range(D):
            gg = g + b

            @pl.when(gg < NCHUNK)
            def _phase():
                cw, cp = gather_copies(gg, b)
                cw.wait()
                cp.wait()

                for h in range(2):
                    @pl.when(gg >= 1)
                    def _():
                        store_copy(gg - 1, h).wait()

                    @pl.loop(0, HH)
                    def _row(r):
                        for u in range(HIDDEN // L):
                            s = pl.ds(u * L, L)
                            obuf[h][r, s] = (wbuf[b][r + h * HH, s]
                                             + pbuf[b][r + h * HH, s])

                    store_copy(gg, h).start()

                @pl.when(gg + D < NCHUNK)
                def _():
                    nw, np_ = gather_copies(gg + D, b)
                    nw.start()
                    np_.start()

    for h in range(2):
        store_copy(NCHUNK - 1, h).wait()


def kernel(input_ids, position_ids, word_embeddings, position_embeddings):
    wids = input_ids.reshape(-1).astype(jnp.int32)
    pids = position_ids.reshape(-1).astype(jnp.int32)
    mesh = plsc.VectorSubcoreMesh(core_axis_name="c", subcore_axis_name="s")
    k = pl.kernel(
        _emb_body,
        out_type=jax.ShapeDtypeStruct((N, HIDDEN), jnp.float32),
        mesh=mesh,
        scratch_types=(
            [pltpu.VMEM((RPW,), jnp.int32)] * 2
            + [pltpu.VMEM((C, HIDDEN), jnp.float32)] * (2 * D)
            + [pltpu.VMEM((C // 2, HIDDEN), jnp.float32)] * 2
            + [pltpu.SemaphoreType.DMA] * (2 * D + 2)
        ),
    )
    out = k(word_embeddings, position_embeddings, wids, pids)
    return out.reshape(input_ids.shape + (HIDDEN,))


# speedup vs baseline: 1.0834x; 1.0369x over previous
import jax
import jax.numpy as jnp
from jax import lax
from jax.experimental import pallas as pl
from jax.experimental.pallas import tpu as pltpu
from jax.experimental.pallas import tpu_sc as plsc

HIDDEN = 1024
N = 4 * 4096
NC, NS, L = 2, 16, 16
NW = NC * NS
RPW = N // NW
C = 16
NCHUNK = RPW // C


def _emb_body(w_hbm, p_hbm, wi_hbm, pi_hbm, o_hbm,
              widx, pidx,
              wb0, wb1, pb0, pb1, ob0, ob1,
              sw0, sw1, sp0, sp1, ss0, ss1):
    wbuf = (wb0, wb1)
    pbuf = (pb0, pb1)
    obuf = (ob0, ob1)
    sem_w = (sw0, sw1)
    sem_p = (sp0, sp1)
    sem_s = (ss0, ss1)

    wid = lax.axis_index("s") * NC + lax.axis_index("c")
    base = wid * RPW
    pltpu.sync_copy(wi_hbm.at[pl.ds(base, RPW)], widx)
    pltpu.sync_copy(pi_hbm.at[pl.ds(base, RPW)], pidx)

    def gather_copies(g, b):
        cw = pltpu.make_async_copy(
            w_hbm.at[widx.at[pl.ds(g * C, C)]], wbuf[b], sem_w[b])
        cp = pltpu.make_async_copy(
            p_hbm.at[pidx.at[pl.ds(g * C, C)]], pbuf[b], sem_p[b])
        return cw, cp

    def store_copy(g, b):
        return pltpu.make_async_copy(
            obuf[b], o_hbm.at[pl.ds(base + g * C, C)], sem_s[b])

    for b in (0, 1):
        cw, cp = gather_copies(b, b)
        cw.start()
        cp.start()

    @pl.loop(0, NCHUNK, step=2)
    def _pair(g):
        for b in (0, 1):
            gg = g + b
            cw, cp = gather_copies(gg, b)
            cw.wait()
            cp.wait()

            @pl.when(gg >= 2)
            def _():
                store_copy(gg - 2, b).wait()

            @pl.loop(0, C)
            def _row(r):
                for u in range(HIDDEN // L):
                    s = pl.ds(u * L, L)
                    obuf[b][r, s] = wbuf[b][r, s] + pbuf[b][r, s]

            @pl.when(gg + 2 < NCHUNK)
            def _():
                nw, np_ = gather_copies(gg + 2, b)
                nw.start()
                np_.start()

            store_copy(gg, b).start()

    for b in (0, 1):
        store_copy(NCHUNK - 2 + b, b).wait()


def kernel(input_ids, position_ids, word_embeddings, position_embeddings):
    wids = input_ids.reshape(-1).astype(jnp.int32)
    pids = position_ids.reshape(-1).astype(jnp.int32)
    mesh = plsc.VectorSubcoreMesh(core_axis_name="c", subcore_axis_name="s")
    k = pl.kernel(
        _emb_body,
        out_type=jax.ShapeDtypeStruct((N, HIDDEN), jnp.float32),
        mesh=mesh,
        scratch_types=(
            [pltpu.VMEM((RPW,), jnp.int32)] * 2
            + [pltpu.VMEM((C, HIDDEN), jnp.float32)] * 6
            + [pltpu.SemaphoreType.DMA] * 6
        ),
    )
    out = k(word_embeddings, position_embeddings, wids, pids)
    return out.reshape(input_ids.shape + (HIDDEN,))


# speedup vs baseline: 1.0835x; 1.0001x over previous
import jax
import jax.numpy as jnp
from jax import lax
from jax.experimental import pallas as pl
from jax.experimental.pallas import tpu as pltpu
from jax.experimental.pallas import tpu_sc as plsc

HIDDEN = 1024
N = 4 * 4096
NC, NS, L = 2, 16, 16
NW = NC * NS
RPW = N // NW
C = 16
NCHUNK = RPW // C


def _emb_body(w_hbm, p_hbm, wi_hbm, pi_hbm, o_hbm,
              widx, pidx,
              wb0, wb1, pb0, pb1, ob0, ob1,
              sw0, sw1, sp0, sp1, ss0, ss1):
    wbuf = (wb0, wb1)
    pbuf = (pb0, pb1)
    obuf = (ob0, ob1)
    sem_w = (sw0, sw1)
    sem_p = (sp0, sp1)
    sem_s = (ss0, ss1)

    wid = lax.axis_index("s") * NC + lax.axis_index("c")
    base = wid * RPW
    pltpu.sync_copy(wi_hbm.at[pl.ds(base, RPW)], widx)
    pltpu.sync_copy(pi_hbm.at[pl.ds(base, RPW)], pidx)

    def gather_copies(g, b):
        copies = []
        for k in (0, 1):
            sub = pl.ds(g * C + k * (C // 2), C // 2)
            dst = pl.ds(k * (C // 2), C // 2)
            copies.append(pltpu.make_async_copy(
                w_hbm.at[widx.at[sub]], wbuf[b].at[dst], sem_w[b]))
            copies.append(pltpu.make_async_copy(
                p_hbm.at[pidx.at[sub]], pbuf[b].at[dst], sem_p[b]))
        return copies

    def store_copy(g, b):
        return pltpu.make_async_copy(
            obuf[b], o_hbm.at[pl.ds(base + g * C, C)], sem_s[b])

    for b in (0, 1):
        for c in gather_copies(b, b):
            c.start()

    @pl.loop(0, NCHUNK, step=2)
    def _pair(g):
        for b in (0, 1):
            gg = g + b
            for c in gather_copies(gg, b):
                c.wait()

            @pl.when(gg >= 2)
            def _():
                store_copy(gg - 2, b).wait()

            @pl.loop(0, C)
            def _row(r):
                for u in range(HIDDEN // L):
                    s = pl.ds(u * L, L)
                    obuf[b][r, s] = wbuf[b][r, s] + pbuf[b][r, s]

            @pl.when(gg + 2 < NCHUNK)
            def _():
                for c in gather_copies(gg + 2, b):
                    c.start()

            store_copy(gg, b).start()

    for b in (0, 1):
        store_copy(NCHUNK - 2 + b, b).wait()


def kernel(input_ids, position_ids, word_embeddings, position_embeddings):
    wids = input_ids.reshape(-1).astype(jnp.int32)
    pids = position_ids.reshape(-1).astype(jnp.int32)
    mesh = plsc.VectorSubcoreMesh(core_axis_name="c", subcore_axis_name="s")
    k = pl.kernel(
        _emb_body,
        out_type=jax.ShapeDtypeStruct((N, HIDDEN), jnp.float32),
        mesh=mesh,
        scratch_types=(
            [pltpu.VMEM((RPW,), jnp.int32)] * 2
            + [pltpu.VMEM((C, HIDDEN), jnp.float32)] * 6
            + [pltpu.SemaphoreType.DMA] * 6
        ),
    )
    out = k(word_embeddings, position_embeddings, wids, pids)
    return out.reshape(input_ids.shape + (HIDDEN,))


# speedup vs baseline: 1.0847x; 1.0011x over previous
import jax
import jax.numpy as jnp
from jax import lax
from jax.experimental import pallas as pl
from jax.experimental.pallas import tpu as pltpu
from jax.experimental.pallas import tpu_sc as plsc

HIDDEN = 1024
N = 4 * 4096
NC, NS, L = 2, 16, 16
NW = NC * NS
RPW = N // NW
C = 16
NCHUNK = RPW // C


def _emb_body(w_hbm, p_hbm, wi_hbm, pi_hbm, o_hbm,
              widx, pidx,
              wb0, wb1, pb0, pb1, ob0, ob1,
              sw0, sw1, sp0, sp1, ss0, ss1):
    wbuf = (wb0, wb1)
    pbuf = (pb0, pb1)
    obuf = (ob0, ob1)
    sem_w = (sw0, sw1)
    sem_p = (sp0, sp1)
    sem_s = (ss0, ss1)

    wid = lax.axis_index("s") * NC + lax.axis_index("c")
    base = wid * RPW
    pltpu.sync_copy(wi_hbm.at[pl.ds(base, RPW)], widx)
    pltpu.sync_copy(pi_hbm.at[pl.ds(base, RPW)], pidx)

    def gather_copies(g, b):
        cw = pltpu.make_async_copy(
            w_hbm.at[widx.at[pl.ds(g * C, C)]], wbuf[b], sem_w[b])
        cp = pltpu.make_async_copy(
            p_hbm.at[pidx.at[pl.ds(g * C, C)]], pbuf[b], sem_p[b])
        return (cw, cp)

    def store_copy(g, b):
        return pltpu.make_async_copy(
            obuf[b], o_hbm.at[pl.ds(base + g * C, C)], sem_s[b])

    for b in (0, 1):
        for c in gather_copies(b, b):
            c.start()

    @pl.loop(0, NCHUNK, step=2)
    def _pair(g):
        for b in (0, 1):
            gg = g + b
            for c in gather_copies(gg, b):
                c.wait()

            @pl.when(gg >= 2)
            def _():
                store_copy(gg - 2, b).wait()

            @pl.loop(0, C)
            def _row(r):
                for u in range(HIDDEN // L):
                    s = pl.ds(u * L, L)
                    obuf[b][r, s] = wbuf[b][r, s] + pbuf[b][r, s]

            @pl.when(gg + 2 < NCHUNK)
            def _():
                for c in gather_copies(gg + 2, b):
                    c.start()

            store_copy(gg, b).start()

    for b in (0, 1):
        store_copy(NCHUNK - 2 + b, b).wait()


def kernel(input_ids, position_ids, word_embeddings, position_embeddings):
    wids = input_ids.reshape(-1).astype(jnp.int32)
    pids = position_ids.reshape(-1).astype(jnp.int32)
    mesh = plsc.VectorSubcoreMesh(core_axis_name="c", subcore_axis_name="s")
    k = pl.kernel(
        _emb_body,
        out_type=jax.ShapeDtypeStruct((N, HIDDEN), jnp.float32),
        mesh=mesh,
        scratch_types=(
            [pltpu.VMEM((RPW,), jnp.int32)] * 2
            + [pltpu.VMEM((C, HIDDEN), jnp.float32)] * 6
            + [pltpu.SemaphoreType.DMA] * 6
        ),
    )
    out = k(word_embeddings, position_embeddings, wids, pids)
    return out.reshape(input_ids.shape + (HIDDEN,))
